# Initial kernel scaffold; baseline (speedup 1.0000x reference)
import jax
import jax.numpy as jnp
from jax.experimental import pallas as pl


def kernel(input, weight0, bias0, knn0, thr0, weight1, bias1, knn1, thr1, weight2, bias2, knn2, thr2, weight3, bias3, knn3, thr3, fc_w, fc_b):
    raise NotImplementedError("write your pallas kernel here")


# speedup vs baseline: 32.3313x; 32.3313x over previous
import functools

import jax
import jax.numpy as jnp
from jax import lax
from jax.experimental import pallas as pl
from jax.experimental.pallas import tpu as pltpu
from jax.experimental.pallas import tpu_sc as plsc

B = 16
T = 20
K = 16
DIMS = (4096, 2048, 1024, 512)
PREV = (8192, 4096, 2048, 1024)
ALPHA = 0.9
BETA = 0.85
NQ = 4
NG_PER_CORE = 4
L = 16


def _body(input_h,
          knnT0, wT0, thr0_h, bias0_h,
          knnT1, wT1, thr1_h, bias1_h,
          knnT2, wT2, thr2_h, bias2_h,
          knnT3, wT3, thr3_h, bias3_h,
          mo0, mo1, mo2, mo3, so0, so1, so2, so3,
          xbuf,
          kv0, wv0, tv0, bv0, sy0, me0, sp0,
          kv1, wv1, tv1, bv1, sy1, me1, sp1,
          kv2, wv2, tv2, bv2, sy2, me2, sp2,
          kv3, wv3, tv3, bv3, sy3, me3, sp3,
          xch0, xch1, xch2):
    c = lax.axis_index("c")
    s = lax.axis_index("s")
    gl = s // NQ
    q = s % NQ
    b0 = (c * NG_PER_CORE + gl) * 2

    knn_h = (knnT0, knnT1, knnT2, knnT3)
    w_h = (wT0, wT1, wT2, wT3)
    thr_h = (thr0_h, thr1_h, thr2_h, thr3_h)
    bias_h = (bias0_h, bias1_h, bias2_h, bias3_h)
    mo = (mo0, mo1, mo2, mo3)
    so = (so0, so1, so2, so3)
    kv = (kv0, kv1, kv2, kv3)
    wv = (wv0, wv1, wv2, wv3)
    tv = (tv0, tv1, tv2, tv3)
    bv = (bv0, bv1, bv2, bv3)
    sy = (sy0, sy1, sy2, sy3)
    me = (me0, me1, me2, me3)
    sp = (sp0, sp1, sp2, sp3)
    xch = (xch0, xch1, xch2)

    zeros16 = jnp.zeros((L,), jnp.float32)
    roff = jnp.full((L,), PREV[0], jnp.int32)

    for i in range(4):
        W = DIMS[i] // NQ
        j0 = q * W
        pltpu.sync_copy(knn_h[i].at[:, pl.ds(j0, W)], kv[i])
        pltpu.sync_copy(w_h[i].at[:, pl.ds(j0, W)], wv[i])
        pltpu.sync_copy(thr_h[i].at[pl.ds(j0, W)], tv[i])
        pltpu.sync_copy(bias_h[i].at[pl.ds(j0, W)], bv[i])

        def zbody(jb, _, i=i):
            o = pl.multiple_of(jb * L, L)
            sy[i][0, pl.ds(o, L)] = zeros16
            sy[i][1, pl.ds(o, L)] = zeros16
            me[i][0, pl.ds(o, L)] = zeros16
            me[i][1, pl.ds(o, L)] = zeros16
            return _
        lax.fori_loop(0, W // L, zbody, None)

    def step(t, carry):
        pltpu.sync_copy(input_h.at[b0, t], xbuf.at[pl.ds(0, PREV[0])])
        pltpu.sync_copy(input_h.at[b0 + 1, t],
                        xbuf.at[pl.ds(PREV[0], PREV[0])])
        for i in range(4):
            W = DIMS[i] // NQ
            kvi, wvi, tvi, bvi = kv[i], wv[i], tv[i], bv[i]
            syi, mei, spi = sy[i], me[i], sp[i]

            def jbody(jb, _, kvi=kvi, wvi=wvi, tvi=tvi, bvi=bvi,
                      syi=syi, mei=mei, spi=spi):
                o = pl.multiple_of(jb * L, L)
                acc0 = bvi[pl.ds(o, L)]
                acc1 = acc0
                for k in range(K):
                    idx = kvi[k, pl.ds(o, L)]
                    wk = wvi[k, pl.ds(o, L)]
                    g0 = plsc.load_gather(xbuf, [idx])
                    g1 = plsc.load_gather(xbuf, [idx + roff])
                    acc0 = acc0 + g0 * wk
                    acc1 = acc1 + g1 * wk
                thrv = tvi[pl.ds(o, L)]
                for r, acc in ((0, acc0), (1, acc1)):
                    m = mei[r, pl.ds(o, L)]
                    sn = ALPHA * syi[r, pl.ds(o, L)] + acc
                    mn = BETA * m + sn - jnp.where(m > thrv, thrv, 0.0)
                    spkv = jnp.where(mn > thrv, 1.0, 0.0)
                    syi[r, pl.ds(o, L)] = sn
                    mei[r, pl.ds(o, L)] = mn
                    spi[r, pl.ds(o, L)] = spkv
                return _
            lax.fori_loop(0, W // L, jbody, None)

            pltpu.sync_copy(mei, mo[i].at[t, pl.ds(b0, 2), pl.ds(q * W, W)])
            pltpu.sync_copy(spi, so[i].at[t, pl.ds(b0, 2), pl.ds(q * W, W)])
            if i < 3:
                pltpu.sync_copy(spi, xch[i].at[pl.ds(2 * gl, 2),
                                               pl.ds(q * W, W)])
                plsc.subcore_barrier()
                pltpu.sync_copy(xch[i].at[2 * gl, :],
                                xbuf.at[pl.ds(0, DIMS[i])])
                pltpu.sync_copy(xch[i].at[2 * gl + 1, :],
                                xbuf.at[pl.ds(PREV[0], DIMS[i])])
        return carry

    lax.fori_loop(0, T, step, None)


@jax.jit
def _run(input, knnT, wT, thr, bias):
    mesh = plsc.VectorSubcoreMesh(core_axis_name="c", subcore_axis_name="s")
    out_type = (
        tuple(jax.ShapeDtypeStruct((T, B, d), jnp.float32) for d in DIMS)
        + tuple(jax.ShapeDtypeStruct((T, B, d), jnp.float32) for d in DIMS)
    )
    scratch = [pltpu.VMEM((2 * PREV[0],), jnp.float32)]
    for d in DIMS:
        W = d // NQ
        scratch += [
            pltpu.VMEM((K, W), jnp.int32),
            pltpu.VMEM((K, W), jnp.float32),
            pltpu.VMEM((W,), jnp.float32),
            pltpu.VMEM((W,), jnp.float32),
            pltpu.VMEM((2, W), jnp.float32),
            pltpu.VMEM((2, W), jnp.float32),
            pltpu.VMEM((2, W), jnp.float32),
        ]
    scratch += [pltpu.VMEM_SHARED((2 * NG_PER_CORE, d), jnp.float32)
                for d in DIMS[:3]]
    flat_in = [input]
    for i in range(4):
        flat_in += [knnT[i], wT[i], thr[i], bias[i]]
    run = pl.kernel(_body, out_type=out_type, mesh=mesh,
                    scratch_types=scratch,
                    compiler_params=pltpu.CompilerParams(
                        needs_layout_passes=False))
    outs = run(*flat_in)
    return outs[:4], outs[4:]


def kernel(input, weight0, bias0, knn0, thr0, weight1, bias1, knn1, thr1,
           weight2, bias2, knn2, thr2, weight3, bias3, knn3, thr3,
           fc_w, fc_b):
    knnT = tuple(k.T.astype(jnp.int32)
                 for k in (knn0, knn1, knn2, knn3))
    wT = tuple(w.T for w in (weight0, weight1, weight2, weight3))
    thr = (thr0, thr1, thr2, thr3)
    bias = tuple(b.reshape(-1) for b in (bias0, bias1, bias2, bias3))
    mem_rec, spk_rec = _run(input, knnT, wT, thr, bias)
    angles = jnp.dot(mem_rec[3][T - 1], fc_w.T) + fc_b
    return tuple(mem_rec) + tuple(spk_rec) + (angles,)


# speedup vs baseline: 42.2949x; 1.3082x over previous
import jax
import jax.numpy as jnp
from jax import lax
from jax.experimental import pallas as pl
from jax.experimental.pallas import tpu as pltpu
from jax.experimental.pallas import tpu_sc as plsc

B = 16
T = 20
K = 16
DIMS = (4096, 2048, 1024, 512)
PREV = (8192, 4096, 2048, 1024)
ALPHA = 0.9
BETA = 0.85
NQ = 4
NG_PER_CORE = 4
L = 16
XROFF = PREV[1]


def _body(input_h,
          knnT0, wT0, thr0_h, bias0_h,
          knnT1, wT1, thr1_h, bias1_h,
          knnT2, wT2, thr2_h, bias2_h,
          knnT3, wT3, thr3_h, bias3_h,
          mo0, mo1, mo2, mo3, so0, so1, so2, so3,
          x0a, x0b, xbuf,
          kv0, wv0, tv0, bv0, sy0, me0, sp0,
          kv1, wv1, tv1, bv1, sy1, me1, sp1,
          kv2, wv2, tv2, bv2, sy2, me2, sp2,
          kv3, wv3, tv3, bv3, sy3, me3, sp3,
          xch0, xch1, xch2,
          msem0, msem1, msem2, msem3, ssem0, ssem1, ssem2, ssem3,
          sem_a, sem_b, xsem):
    c = lax.axis_index("c")
    s = lax.axis_index("s")
    gl = s // NQ
    q = s % NQ
    b0 = (c * NG_PER_CORE + gl) * 2

    knn_h = (knnT0, knnT1, knnT2, knnT3)
    w_h = (wT0, wT1, wT2, wT3)
    thr_h = (thr0_h, thr1_h, thr2_h, thr3_h)
    bias_h = (bias0_h, bias1_h, bias2_h, bias3_h)
    mo = (mo0, mo1, mo2, mo3)
    so = (so0, so1, so2, so3)
    kv = (kv0, kv1, kv2, kv3)
    wv = (wv0, wv1, wv2, wv3)
    tv = (tv0, tv1, tv2, tv3)
    bv = (bv0, bv1, bv2, bv3)
    sy = (sy0, sy1, sy2, sy3)
    me = (me0, me1, me2, me3)
    sp = (sp0, sp1, sp2, sp3)
    xch = (xch0, xch1, xch2)
    msem = (msem0, msem1, msem2, msem3)
    ssem = (ssem0, ssem1, ssem2, ssem3)

    zeros16 = jnp.zeros((L,), jnp.float32)

    def fetch_x0(t, buf, sem):
        pltpu.async_copy(input_h.at[b0, t], buf.at[pl.ds(0, PREV[0])], sem)
        pltpu.async_copy(input_h.at[b0 + 1, t],
                         buf.at[pl.ds(PREV[0], PREV[0])], sem)

    def wait_x0(buf, sem):
        pltpu.make_async_copy(input_h.at[b0, 0],
                              buf.at[pl.ds(0, PREV[0])], sem).wait()
        pltpu.make_async_copy(input_h.at[b0, 0],
                              buf.at[pl.ds(PREV[0], PREV[0])], sem).wait()

    def out_slice(o, i):
        W = DIMS[i] // NQ
        return o.at[0, pl.ds(b0, 2), pl.ds(q * W, W)]

    for i in range(4):
        W = DIMS[i] // NQ
        j0 = q * W
        pltpu.sync_copy(knn_h[i].at[:, pl.ds(j0, W)], kv[i])
        pltpu.sync_copy(w_h[i].at[:, pl.ds(j0, W)], wv[i])
        pltpu.sync_copy(thr_h[i].at[pl.ds(j0, W)], tv[i])
        pltpu.sync_copy(bias_h[i].at[pl.ds(j0, W)], bv[i])

        def zbody(jb, carry, i=i):
            o = pl.multiple_of(jb * L, L)
            sy[i][0, pl.ds(o, L)] = zeros16
            sy[i][1, pl.ds(o, L)] = zeros16
            me[i][0, pl.ds(o, L)] = zeros16
            me[i][1, pl.ds(o, L)] = zeros16
            return carry
        lax.fori_loop(0, W // L, zbody, None)
        pltpu.async_copy(me[i], out_slice(mo[i], i), msem[i])
        pltpu.async_copy(sp[i], out_slice(so[i], i), ssem[i])
    fetch_x0(0, x0a, sem_a)

    def do_layer(i, t, x0buf):
        W = DIMS[i] // NQ
        kvi, wvi, tvi, bvi = kv[i], wv[i], tv[i], bv[i]
        syi, mei, spi = sy[i], me[i], sp[i]
        if i == 0:
            src, roff = x0buf, PREV[0]
        else:
            src, roff = xbuf, XROFF
        roffv = jnp.full((L,), roff, jnp.int32)

        pltpu.make_async_copy(mei, out_slice(mo[i], i), msem[i]).wait()
        pltpu.make_async_copy(spi, out_slice(so[i], i), ssem[i]).wait()

        def jbody(jb, carry):
            o = pl.multiple_of(jb * L, L)
            acc0 = bvi[pl.ds(o, L)]
            acc1 = acc0
            for k in range(K):
                idx = kvi[k, pl.ds(o, L)]
                wk = wvi[k, pl.ds(o, L)]
                g0 = plsc.load_gather(src, [idx])
                g1 = plsc.load_gather(src, [idx + roffv])
                acc0 = acc0 + g0 * wk
                acc1 = acc1 + g1 * wk
            thrv = tvi[pl.ds(o, L)]
            for r, acc in ((0, acc0), (1, acc1)):
                m = mei[r, pl.ds(o, L)]
                sn = ALPHA * syi[r, pl.ds(o, L)] + acc
                mn = BETA * m + sn - jnp.where(m > thrv, thrv, 0.0)
                spkv = jnp.where(mn > thrv, 1.0, 0.0)
                syi[r, pl.ds(o, L)] = sn
                mei[r, pl.ds(o, L)] = mn
                spi[r, pl.ds(o, L)] = spkv
            return carry
        lax.fori_loop(0, W // L, jbody, None)

        pltpu.async_copy(mei, mo[i].at[t, pl.ds(b0, 2), pl.ds(q * W, W)],
                         msem[i])
        pltpu.async_copy(spi, so[i].at[t, pl.ds(b0, 2), pl.ds(q * W, W)],
                         ssem[i])
        if i < 3:
            pltpu.sync_copy(spi, xch[i].at[pl.ds(2 * gl, 2),
                                           pl.ds(q * W, W)])
            plsc.subcore_barrier()
            h1 = pltpu.async_copy(xch[i].at[2 * gl, :],
                                  xbuf.at[pl.ds(0, DIMS[i])], xsem)
            h2 = pltpu.async_copy(xch[i].at[2 * gl + 1, :],
                                  xbuf.at[pl.ds(XROFF, DIMS[i])], xsem)
            h1.wait()
            h2.wait()

    def pair(p, carry):
        t0 = p * 2
        fetch_x0(t0 + 1, x0b, sem_b)
        wait_x0(x0a, sem_a)
        for i in range(4):
            do_layer(i, t0, x0a)
        fetch_x0(jnp.minimum(t0 + 2, T - 1), x0a, sem_a)
        wait_x0(x0b, sem_b)
        for i in range(4):
            do_layer(i, t0 + 1, x0b)
        return carry

    lax.fori_loop(0, T // 2, pair, None)

    wait_x0(x0a, sem_a)
    for i in range(4):
        pltpu.make_async_copy(me[i], out_slice(mo[i], i), msem[i]).wait()
        pltpu.make_async_copy(sp[i], out_slice(so[i], i), ssem[i]).wait()


@jax.jit
def _run(input, knnT, wT, thr, bias):
    mesh = plsc.VectorSubcoreMesh(core_axis_name="c", subcore_axis_name="s")
    out_type = (
        tuple(jax.ShapeDtypeStruct((T, B, d), jnp.float32) for d in DIMS)
        + tuple(jax.ShapeDtypeStruct((T, B, d), jnp.float32) for d in DIMS)
    )
    scratch = [
        pltpu.VMEM((2 * PREV[0],), jnp.float32),
        pltpu.VMEM((2 * PREV[0],), jnp.float32),
        pltpu.VMEM((2 * XROFF,), jnp.float32),
    ]
    for d in DIMS:
        W = d // NQ
        scratch += [
            pltpu.VMEM((K, W), jnp.int32),
            pltpu.VMEM((K, W), jnp.float32),
            pltpu.VMEM((W,), jnp.float32),
            pltpu.VMEM((W,), jnp.float32),
            pltpu.VMEM((2, W), jnp.float32),
            pltpu.VMEM((2, W), jnp.float32),
            pltpu.VMEM((2, W), jnp.float32),
        ]
    scratch += [pltpu.VMEM_SHARED((2 * NG_PER_CORE, d), jnp.float32)
                for d in DIMS[:3]]
    scratch += [pltpu.SemaphoreType.DMA] * 11
    flat_in = [input]
    for i in range(4):
        flat_in += [knnT[i], wT[i], thr[i], bias[i]]
    run = pl.kernel(_body, out_type=out_type, mesh=mesh,
                    scratch_types=scratch,
                    compiler_params=pltpu.CompilerParams(
                        needs_layout_passes=False))
    outs = run(*flat_in)
    return outs[:4], outs[4:]


def kernel(input, weight0, bias0, knn0, thr0, weight1, bias1, knn1, thr1,
           weight2, bias2, knn2, thr2, weight3, bias3, knn3, thr3,
           fc_w, fc_b):
    knnT = tuple(k.T.astype(jnp.int32)
                 for k in (knn0, knn1, knn2, knn3))
    wT = tuple(w.T for w in (weight0, weight1, weight2, weight3))
    thr = (thr0, thr1, thr2, thr3)
    bias = tuple(b.reshape(-1) for b in (bias0, bias1, bias2, bias3))
    mem_rec, spk_rec = _run(input, knnT, wT, thr, bias)
    angles = jnp.dot(mem_rec[3][T - 1], fc_w.T) + fc_b
    return tuple(mem_rec) + tuple(spk_rec) + (angles,)


# speedup vs baseline: 47.3287x; 1.1190x over previous
import jax
import jax.numpy as jnp
from jax import lax
from jax.experimental import pallas as pl
from jax.experimental.pallas import tpu as pltpu
from jax.experimental.pallas import tpu_sc as plsc

B = 16
T = 20
K = 16
DIMS = (4096, 2048, 1024, 512)
PREV = (8192, 4096, 2048, 1024)
ALPHA = 0.9
BETA = 0.85
NQ = 4
NG_PER_CORE = 4
L = 16
XROFF = PREV[1]


def _body(input_h,
          knnT0, wT0, thr0_h, bias0_h,
          knnT1, wT1, thr1_h, bias1_h,
          knnT2, wT2, thr2_h, bias2_h,
          knnT3, wT3, thr3_h, bias3_h,
          mo0, mo1, mo2, mo3, so0, so1, so2, so3,
          x0a, x0b, xpbuf,
          kv0, wv0, tv0, bv0, sy0, me0, sp0,
          kv1, wv1, tv1, bv1, sy1, me1, sp1,
          kv2, wv2, tv2, bv2, sy2, me2, sp2,
          kv3, wv3, tv3, bv3, sy3, me3, sp3,
          spp0, spp1, spp2,
          xch0, xch1, xch2,
          msem0, msem1, msem2, msem3, ssem0, ssem1, ssem2, ssem3,
          sem_a, sem_b, xsem):
    c = lax.axis_index("c")
    s = lax.axis_index("s")
    gl = s // NQ
    q = s % NQ
    b0 = (c * NG_PER_CORE + gl) * 2

    knn_h = (knnT0, knnT1, knnT2, knnT3)
    w_h = (wT0, wT1, wT2, wT3)
    thr_h = (thr0_h, thr1_h, thr2_h, thr3_h)
    bias_h = (bias0_h, bias1_h, bias2_h, bias3_h)
    mo = (mo0, mo1, mo2, mo3)
    so = (so0, so1, so2, so3)
    kv = (kv0, kv1, kv2, kv3)
    wv = (wv0, wv1, wv2, wv3)
    tv = (tv0, tv1, tv2, tv3)
    bv = (bv0, bv1, bv2, bv3)
    sy = (sy0, sy1, sy2, sy3)
    me = (me0, me1, me2, me3)
    sp = (sp0, sp1, sp2, sp3)
    spp = (spp0, spp1, spp2)
    xch = (xch0, xch1, xch2)
    msem = (msem0, msem1, msem2, msem3)
    ssem = (ssem0, ssem1, ssem2, ssem3)

    zeros16 = jnp.zeros((L,), jnp.float32)

    def fetch_x0(t, buf, sem):
        pltpu.async_copy(input_h.at[b0, t], buf.at[pl.ds(0, PREV[0])], sem)
        pltpu.async_copy(input_h.at[b0 + 1, t],
                         buf.at[pl.ds(PREV[0], PREV[0])], sem)

    def wait_x0(buf, sem):
        pltpu.make_async_copy(input_h.at[b0, 0],
                              buf.at[pl.ds(0, PREV[0])], sem).wait()
        pltpu.make_async_copy(input_h.at[b0, 0],
                              buf.at[pl.ds(PREV[0], PREV[0])], sem).wait()

    def out_slice(o, i):
        W = DIMS[i] // NQ
        return o.at[0, pl.ds(b0, 2), pl.ds(q * W, W)]

    for i in range(4):
        W = DIMS[i] // NQ
        j0 = q * W
        pltpu.sync_copy(knn_h[i].at[:, pl.ds(j0, W)], kv[i])
        pltpu.sync_copy(w_h[i].at[:, pl.ds(j0, W)], wv[i])
        pltpu.sync_copy(thr_h[i].at[pl.ds(j0, W)], tv[i])
        pltpu.sync_copy(bias_h[i].at[pl.ds(j0, W)], bv[i])

        def zbody(jb, carry, i=i):
            o = pl.multiple_of(jb * L, L)
            sy[i][0, pl.ds(o, L)] = zeros16
            sy[i][1, pl.ds(o, L)] = zeros16
            me[i][0, pl.ds(o, L)] = zeros16
            me[i][1, pl.ds(o, L)] = zeros16
            return carry
        lax.fori_loop(0, W // L, zbody, None)
        pltpu.async_copy(me[i], out_slice(mo[i], i), msem[i])
        pltpu.async_copy(sp[i], out_slice(so[i], i), ssem[i])
    fetch_x0(0, x0a, sem_a)

    def do_layer(i, t, x0buf):
        W = DIMS[i] // NQ
        kvi, wvi, tvi, bvi = kv[i], wv[i], tv[i], bv[i]
        syi, mei, spi = sy[i], me[i], sp[i]
        roffv = jnp.full((L,), PREV[0], jnp.int32)

        pltpu.make_async_copy(mei, out_slice(mo[i], i), msem[i]).wait()
        pltpu.make_async_copy(spi, out_slice(so[i], i), ssem[i]).wait()

        def jbody(jb, carry):
            o = pl.multiple_of(jb * L, L)
            acc0 = bvi[pl.ds(o, L)]
            acc1 = acc0
            for kp in range(K // 2):
                ab = plsc.bitcast(kvi[kp, pl.ds(o, L)], jnp.int16)
                ia, ib = plsc.unpack(ab, format=plsc.PackFormat.INTERLEAVED)
                for k, idx in ((2 * kp, ia), (2 * kp + 1, ib)):
                    wk = wvi[k, pl.ds(o, L)]
                    if i == 0:
                        g0 = plsc.load_gather(x0buf, [idx])
                        g1 = plsc.load_gather(x0buf, [idx + roffv])
                    else:
                        gp = plsc.load_gather(xpbuf, [idx])
                        gb = plsc.bitcast(gp, jnp.bfloat16)
                        g0, g1 = plsc.unpack(
                            gb, format=plsc.PackFormat.INTERLEAVED)
                    acc0 = acc0 + g0 * wk
                    acc1 = acc1 + g1 * wk
            thrv = tvi[pl.ds(o, L)]
            spks = []
            for r, acc in ((0, acc0), (1, acc1)):
                m = mei[r, pl.ds(o, L)]
                sn = ALPHA * syi[r, pl.ds(o, L)] + acc
                mn = BETA * m + sn - jnp.where(m > thrv, thrv, 0.0)
                spkv = jnp.where(mn > thrv, 1.0, 0.0)
                syi[r, pl.ds(o, L)] = sn
                mei[r, pl.ds(o, L)] = mn
                spi[r, pl.ds(o, L)] = spkv
                spks.append(spkv)
            if i < 3:
                pk = plsc.pack(spks[0], spks[1],
                               format=plsc.PackFormat.INTERLEAVED)
                spp[i][pl.ds(o, L)] = plsc.bitcast(pk, jnp.int32)
            return carry
        lax.fori_loop(0, W // L, jbody, None)

        pltpu.async_copy(mei, mo[i].at[t, pl.ds(b0, 2), pl.ds(q * W, W)],
                         msem[i])
        pltpu.async_copy(spi, so[i].at[t, pl.ds(b0, 2), pl.ds(q * W, W)],
                         ssem[i])
        if i < 3:
            pltpu.sync_copy(spp[i], xch[i].at[gl, pl.ds(q * W, W)])
            plsc.subcore_barrier()
            pltpu.async_copy(xch[i].at[gl, :],
                             xpbuf.at[pl.ds(0, DIMS[i])], xsem).wait()

    def pair(p, carry):
        t0 = p * 2
        fetch_x0(t0 + 1, x0b, sem_b)
        wait_x0(x0a, sem_a)
        for i in range(4):
            do_layer(i, t0, x0a)
        fetch_x0(jnp.minimum(t0 + 2, T - 1), x0a, sem_a)
        wait_x0(x0b, sem_b)
        for i in range(4):
            do_layer(i, t0 + 1, x0b)
        return carry

    lax.fori_loop(0, T // 2, pair, None)

    wait_x0(x0a, sem_a)
    for i in range(4):
        pltpu.make_async_copy(me[i], out_slice(mo[i], i), msem[i]).wait()
        pltpu.make_async_copy(sp[i], out_slice(so[i], i), ssem[i]).wait()


@jax.jit
def _run(input, knnT, wT, thr, bias):
    mesh = plsc.VectorSubcoreMesh(core_axis_name="c", subcore_axis_name="s")
    out_type = (
        tuple(jax.ShapeDtypeStruct((T, B, d), jnp.float32) for d in DIMS)
        + tuple(jax.ShapeDtypeStruct((T, B, d), jnp.float32) for d in DIMS)
    )
    scratch = [
        pltpu.VMEM((2 * PREV[0],), jnp.float32),
        pltpu.VMEM((2 * PREV[0],), jnp.float32),
        pltpu.VMEM((XROFF,), jnp.int32),
    ]
    for d in DIMS:
        W = d // NQ
        scratch += [
            pltpu.VMEM((K // 2, W), jnp.int32),
            pltpu.VMEM((K, W), jnp.float32),
            pltpu.VMEM((W,), jnp.float32),
            pltpu.VMEM((W,), jnp.float32),
            pltpu.VMEM((2, W), jnp.float32),
            pltpu.VMEM((2, W), jnp.float32),
            pltpu.VMEM((2, W), jnp.float32),
        ]
    scratch += [pltpu.VMEM((d // NQ,), jnp.int32) for d in DIMS[:3]]
    scratch += [pltpu.VMEM_SHARED((NG_PER_CORE, d), jnp.int32)
                for d in DIMS[:3]]
    scratch += [pltpu.SemaphoreType.DMA] * 11
    flat_in = [input]
    for i in range(4):
        flat_in += [knnT[i], wT[i], thr[i], bias[i]]
    run = pl.kernel(_body, out_type=out_type, mesh=mesh,
                    scratch_types=scratch,
                    compiler_params=pltpu.CompilerParams(
                        needs_layout_passes=False))
    outs = run(*flat_in)
    return outs[:4], outs[4:]


def kernel(input, weight0, bias0, knn0, thr0, weight1, bias1, knn1, thr1,
           weight2, bias2, knn2, thr2, weight3, bias3, knn3, thr3,
           fc_w, fc_b):
    def _pack_idx(k):
        kT = k.T.astype(jnp.int32)
        return kT[0::2] | (kT[1::2] << 16)
    knnT = tuple(_pack_idx(k) for k in (knn0, knn1, knn2, knn3))
    wT = tuple(w.T for w in (weight0, weight1, weight2, weight3))
    thr = (thr0, thr1, thr2, thr3)
    bias = tuple(b.reshape(-1) for b in (bias0, bias1, bias2, bias3))
    mem_rec, spk_rec = _run(input, knnT, wT, thr, bias)
    angles = jnp.dot(mem_rec[3][T - 1], fc_w.T) + fc_b
    return tuple(mem_rec) + tuple(spk_rec) + (angles,)


# speedup vs baseline: 47.6239x; 1.0062x over previous
import jax
import jax.numpy as jnp
from jax import lax
from jax.experimental import pallas as pl
from jax.experimental.pallas import tpu as pltpu
from jax.experimental.pallas import tpu_sc as plsc

B = 16
T = 20
K = 16
DIMS = (4096, 2048, 1024, 512)
PREV = (8192, 4096, 2048, 1024)
ALPHA = 0.9
BETA = 0.85
NQ = 8
NR = 4
NG_PER_CORE = 2
L = 16
ILV = plsc.PackFormat.INTERLEAVED
WCH = (512, 256, 128, 128)


def _body(input_h,
          knnT0, wT0, thr0_h, bias0_h,
          knnT1, wT1, thr1_h, bias1_h,
          knnT2, wT2, thr2_h, bias2_h,
          knnT3, wT3, thr3_h, bias3_h,
          mo0, mo1, mo2, mo3, so0, so1, so2, so3,
          x0a, x0b, xp0, xp1,
          kv0, wv0, tv0, bv0, sy0, me0, sp0,
          kv1, wv1, tv1, bv1, sy1, me1, sp1,
          kv2, wv2, tv2, bv2, sy2, me2, sp2,
          kv3, wv3, tv3, bv3, sy3, me3, sp3,
          spp00, spp01, spp10, spp11, spp20, spp21,
          xch0, xch1, xch2,
          msem0, msem1, msem2, msem3, ssem0, ssem1, ssem2, ssem3,
          sem_a, sem_b, xsem):
    c = lax.axis_index("c")
    s = lax.axis_index("s")
    gl = s // NQ
    q = s % NQ
    b0 = (c * NG_PER_CORE + gl) * NR

    knn_h = (knnT0, knnT1, knnT2, knnT3)
    w_h = (wT0, wT1, wT2, wT3)
    thr_h = (thr0_h, thr1_h, thr2_h, thr3_h)
    bias_h = (bias0_h, bias1_h, bias2_h, bias3_h)
    mo = (mo0, mo1, mo2, mo3)
    so = (so0, so1, so2, so3)
    kv = (kv0, kv1, kv2, kv3)
    wv = (wv0, wv1, wv2, wv3)
    tv = (tv0, tv1, tv2, tv3)
    bv = (bv0, bv1, bv2, bv3)
    sy = (sy0, sy1, sy2, sy3)
    me = (me0, me1, me2, me3)
    sp = (sp0, sp1, sp2, sp3)
    spp = ((spp00, spp01), (spp10, spp11), (spp20, spp21))
    xch = (xch0, xch1, xch2)
    xp = (xp0, xp1)
    msem = (msem0, msem1, msem2, msem3)
    ssem = (ssem0, ssem1, ssem2, ssem3)

    zeros16 = jnp.zeros((L,), jnp.float32)
    roffv = tuple(jnp.full((L,), r * PREV[0], jnp.int32)
                  for r in range(1, NR))

    def fetch_x0(t, buf, sem):
        for r in range(NR):
            pltpu.async_copy(input_h.at[b0 + r, t],
                             buf.at[pl.ds(r * PREV[0], PREV[0])], sem)

    def wait_x0(buf, sem):
        for r in range(NR):
            pltpu.make_async_copy(input_h.at[b0, 0],
                                  buf.at[pl.ds(r * PREV[0], PREV[0])],
                                  sem).wait()

    def out_slice(o, i):
        W = WCH[i]
        return o.at[0, pl.ds(b0, NR), pl.ds(q * W, W)]

    for i in range(4):
        W = WCH[i]

        def prolog(i=i, W=W):
            j0 = q * W
            pltpu.sync_copy(knn_h[i].at[:, pl.ds(j0, W)], kv[i])
            pltpu.sync_copy(w_h[i].at[:, pl.ds(j0, W)], wv[i])
            pltpu.sync_copy(thr_h[i].at[pl.ds(j0, W)], tv[i])
            pltpu.sync_copy(bias_h[i].at[pl.ds(j0, W)], bv[i])

            def zbody(jb, carry):
                o = pl.multiple_of(jb * L, L)
                for r in range(NR):
                    sy[i][r, pl.ds(o, L)] = zeros16
                    me[i][r, pl.ds(o, L)] = zeros16
                return carry
            lax.fori_loop(0, W // L, zbody, None)
            pltpu.async_copy(me[i], out_slice(mo[i], i), msem[i])
            pltpu.async_copy(sp[i], out_slice(so[i], i), ssem[i])
        if i == 3:
            pl.when(q < 4)(prolog)
        else:
            prolog()
    fetch_x0(0, x0a, sem_a)

    def do_layer(i, t, x0buf):
        W = WCH[i]
        kvi, wvi, tvi, bvi = kv[i], wv[i], tv[i], bv[i]
        syi, mei, spi = sy[i], me[i], sp[i]

        pltpu.make_async_copy(mei, out_slice(mo[i], i), msem[i]).wait()
        pltpu.make_async_copy(spi, out_slice(so[i], i), ssem[i]).wait()

        def jbody(jb, carry):
            o = pl.multiple_of(jb * L, L)
            b = bvi[pl.ds(o, L)]
            acc = [b, b, b, b]
            for kp in range(K // 2):
                ab = plsc.bitcast(kvi[kp, pl.ds(o, L)], jnp.int16)
                ia, ib = plsc.unpack(ab, format=ILV)
                for k, idx in ((2 * kp, ia), (2 * kp + 1, ib)):
                    wk = wvi[k, pl.ds(o, L)]
                    if i == 0:
                        g = [plsc.load_gather(x0buf, [idx])]
                        g += [plsc.load_gather(x0buf, [idx + roffv[r - 1]])
                              for r in range(1, NR)]
                    else:
                        g = []
                        for p in range(NR // 2):
                            gp = plsc.load_gather(xp[p], [idx])
                            gb = plsc.bitcast(gp, jnp.bfloat16)
                            g += list(plsc.unpack(gb, format=ILV))
                    for r in range(NR):
                        acc[r] = acc[r] + g[r] * wk
            thrv = tvi[pl.ds(o, L)]
            spks = []
            for r in range(NR):
                m = mei[r, pl.ds(o, L)]
                sn = ALPHA * syi[r, pl.ds(o, L)] + acc[r]
                mn = BETA * m + sn - jnp.where(m > thrv, thrv, 0.0)
                spkv = jnp.where(mn > thrv, 1.0, 0.0)
                syi[r, pl.ds(o, L)] = sn
                mei[r, pl.ds(o, L)] = mn
                spi[r, pl.ds(o, L)] = spkv
                spks.append(spkv)
            if i < 3:
                for p in range(NR // 2):
                    pk = plsc.pack(spks[2 * p], spks[2 * p + 1], format=ILV)
                    spp[i][p][pl.ds(o, L)] = plsc.bitcast(pk, jnp.int32)
            return carry
        lax.fori_loop(0, W // L, jbody, None)

        pltpu.async_copy(mei, mo[i].at[t, pl.ds(b0, NR), pl.ds(q * W, W)],
                         msem[i])
        pltpu.async_copy(spi, so[i].at[t, pl.ds(b0, NR), pl.ds(q * W, W)],
                         ssem[i])
        if i < 3:
            for p in range(NR // 2):
                pltpu.sync_copy(spp[i][p],
                                xch[i].at[gl * (NR // 2) + p,
                                          pl.ds(q * W, W)])
            plsc.subcore_barrier()
            hs = [pltpu.async_copy(xch[i].at[gl * (NR // 2) + p, :],
                                   xp[p].at[pl.ds(0, DIMS[i])], xsem)
                  for p in range(NR // 2)]
            for h in hs:
                h.wait()

    def pair(pidx, carry):
        t0 = pidx * 2
        fetch_x0(t0 + 1, x0b, sem_b)
        wait_x0(x0a, sem_a)
        for i in range(3):
            do_layer(i, t0, x0a)
        pl.when(q < 4)(lambda: do_layer(3, t0, x0a))
        fetch_x0(jnp.minimum(t0 + 2, T - 1), x0a, sem_a)
        wait_x0(x0b, sem_b)
        for i in range(3):
            do_layer(i, t0 + 1, x0b)
        pl.when(q < 4)(lambda: do_layer(3, t0 + 1, x0b))
        return carry

    lax.fori_loop(0, T // 2, pair, None)

    wait_x0(x0a, sem_a)
    for i in range(4):
        def drain(i=i):
            pltpu.make_async_copy(me[i], out_slice(mo[i], i),
                                  msem[i]).wait()
            pltpu.make_async_copy(sp[i], out_slice(so[i], i),
                                  ssem[i]).wait()
        if i == 3:
            pl.when(q < 4)(drain)
        else:
            drain()


@jax.jit
def _run(input, knnT, wT, thr, bias):
    mesh = plsc.VectorSubcoreMesh(core_axis_name="c", subcore_axis_name="s")
    out_type = (
        tuple(jax.ShapeDtypeStruct((T, B, d), jnp.float32) for d in DIMS)
        + tuple(jax.ShapeDtypeStruct((T, B, d), jnp.float32) for d in DIMS)
    )
    scratch = [
        pltpu.VMEM((NR * PREV[0],), jnp.float32),
        pltpu.VMEM((NR * PREV[0],), jnp.float32),
        pltpu.VMEM((PREV[1],), jnp.int32),
        pltpu.VMEM((PREV[1],), jnp.int32),
    ]
    for W in WCH:
        scratch += [
            pltpu.VMEM((K // 2, W), jnp.int32),
            pltpu.VMEM((K, W), jnp.float32),
            pltpu.VMEM((W,), jnp.float32),
            pltpu.VMEM((W,), jnp.float32),
            pltpu.VMEM((NR, W), jnp.float32),
            pltpu.VMEM((NR, W), jnp.float32),
            pltpu.VMEM((NR, W), jnp.float32),
        ]
    for W in WCH[:3]:
        scratch += [pltpu.VMEM((W,), jnp.int32)] * (NR // 2)
    scratch += [pltpu.VMEM_SHARED((NG_PER_CORE * (NR // 2), d), jnp.int32)
                for d in DIMS[:3]]
    scratch += [pltpu.SemaphoreType.DMA] * 11
    flat_in = [input]
    for i in range(4):
        flat_in += [knnT[i], wT[i], thr[i], bias[i]]
    run = pl.kernel(_body, out_type=out_type, mesh=mesh,
                    scratch_types=scratch,
                    compiler_params=pltpu.CompilerParams(
                        needs_layout_passes=False))
    outs = run(*flat_in)
    return outs[:4], outs[4:]


def kernel(input, weight0, bias0, knn0, thr0, weight1, bias1, knn1, thr1,
           weight2, bias2, knn2, thr2, weight3, bias3, knn3, thr3,
           fc_w, fc_b):
    def _pack_idx(k):
        kT = k.T.astype(jnp.int32)
        return kT[0::2] | (kT[1::2] << 16)
    knnT = tuple(_pack_idx(k) for k in (knn0, knn1, knn2, knn3))
    wT = tuple(w.T for w in (weight0, weight1, weight2, weight3))
    thr = (thr0, thr1, thr2, thr3)
    bias = tuple(b.reshape(-1) for b in (bias0, bias1, bias2, bias3))
    mem_rec, spk_rec = _run(input, knnT, wT, thr, bias)
    angles = jnp.dot(mem_rec[3][T - 1], fc_w.T) + fc_b
    return tuple(mem_rec) + tuple(spk_rec) + (angles,)


# speedup vs baseline: 51.3304x; 1.0778x over previous
import jax
import jax.numpy as jnp
from jax import lax
from jax.experimental import pallas as pl
from jax.experimental.pallas import tpu as pltpu
from jax.experimental.pallas import tpu_sc as plsc

B = 16
T = 20
K = 16
DIMS = (4096, 2048, 1024, 512)
PREV = (8192, 4096, 2048, 1024)
ALPHA = 0.9
BETA = 0.85
NQ = 8
NR = 4
NP = NR // 2
NG_PER_CORE = 2
L = 16
ILV = plsc.PackFormat.INTERLEAVED
WCH = (512, 256, 128, 128)
NSTAGE = T + 4


def _body(input_h,
          knnT0, wT0, thr0_h, bias0_h,
          knnT1, wT1, thr1_h, bias1_h,
          knnT2, wT2, thr2_h, bias2_h,
          knnT3, wT3, thr3_h, bias3_h,
          mo0, mo1, mo2, mo3, so0, so1, so2, so3,
          x0a, x0b,
          xp1a, xp1b, xp2a, xp2b, xp3a, xp3b,
          kv0, wv0, tv0, bv0, sy0, me0, sp0,
          kv1, wv1, tv1, bv1, sy1, me1, sp1,
          kv2, wv2, tv2, bv2, sy2, me2, sp2,
          kv3, wv3, tv3, bv3, sy3, me3, sp3,
          spp00, spp01, spp10, spp11, spp20, spp21,
          xcA0, xcA1, xcA2, xcB0, xcB1, xcB2,
          msem0, msem1, msem2, msem3, ssem0, ssem1, ssem2, ssem3,
          sem_a, sem_b, xsem, psem):
    c = lax.axis_index("c")
    s_id = lax.axis_index("s")
    gl = s_id // NQ
    q = s_id % NQ
    b0 = (c * NG_PER_CORE + gl) * NR

    knn_h = (knnT0, knnT1, knnT2, knnT3)
    w_h = (wT0, wT1, wT2, wT3)
    thr_h = (thr0_h, thr1_h, thr2_h, thr3_h)
    bias_h = (bias0_h, bias1_h, bias2_h, bias3_h)
    mo = (mo0, mo1, mo2, mo3)
    so = (so0, so1, so2, so3)
    kv = (kv0, kv1, kv2, kv3)
    wv = (wv0, wv1, wv2, wv3)
    tv = (tv0, tv1, tv2, tv3)
    bv = (bv0, bv1, bv2, bv3)
    sy = (sy0, sy1, sy2, sy3)
    me = (me0, me1, me2, me3)
    sp = (sp0, sp1, sp2, sp3)
    spp = ((spp00, spp01), (spp10, spp11), (spp20, spp21))
    xcA = (xcA0, xcA1, xcA2)
    xcB = (xcB0, xcB1, xcB2)
    xpl = {1: (xp1a, xp1b), 2: (xp2a, xp2b), 3: (xp3a, xp3b)}
    msem = (msem0, msem1, msem2, msem3)
    ssem = (ssem0, ssem1, ssem2, ssem3)

    zeros16 = jnp.zeros((L,), jnp.float32)
    roffv = tuple(jnp.full((L,), r * PREV[0], jnp.int32)
                  for r in range(1, NR))

    def fetch_x0(t, buf, sem):
        for r in range(NR):
            pltpu.async_copy(input_h.at[b0 + r, t],
                             buf.at[pl.ds(r * PREV[0], PREV[0])], sem)

    def wait_x0(buf, sem):
        for r in range(NR):
            pltpu.make_async_copy(input_h.at[b0, 0],
                                  buf.at[pl.ds(r * PREV[0], PREV[0])],
                                  sem).wait()

    def out_slice(o, i):
        W = WCH[i]
        return o.at[0, pl.ds(b0, NR), pl.ds(q * W, W)]

    def fire_reads(xc):
        for i in (1, 2, 3):
            d = PREV[i]
            for p in range(NP):
                pltpu.async_copy(xc[i - 1].at[gl * NP + p, :],
                                 xpl[i][p].at[pl.ds(0, d)], xsem)

    def wait_reads(xc):
        for i in (1, 2, 3):
            d = PREV[i]
            for p in range(NP):
                pltpu.make_async_copy(xc[i - 1].at[gl * NP + p, :],
                                      xpl[i][p].at[pl.ds(0, d)],
                                      xsem).wait()

    for i in range(4):
        W = WCH[i]

        def prolog(i=i, W=W):
            j0 = q * W
            pltpu.sync_copy(knn_h[i].at[:, pl.ds(j0, W)], kv[i])
            pltpu.sync_copy(w_h[i].at[:, pl.ds(j0, W)], wv[i])
            pltpu.sync_copy(thr_h[i].at[pl.ds(j0, W)], tv[i])
            pltpu.sync_copy(bias_h[i].at[pl.ds(j0, W)], bv[i])

            def zbody(jb, carry):
                o = pl.multiple_of(jb * L, L)
                for r in range(NR):
                    sy[i][r, pl.ds(o, L)] = zeros16
                    me[i][r, pl.ds(o, L)] = zeros16
                return carry
            lax.fori_loop(0, W // L, zbody, None)
            pltpu.async_copy(me[i], out_slice(mo[i], i), msem[i])
            pltpu.async_copy(sp[i], out_slice(so[i], i), ssem[i])
        if i == 3:
            pl.when(q < 4)(prolog)
        else:
            prolog()
    fetch_x0(0, x0a, sem_a)
    fire_reads(xcB)

    def do_layer(i, t, x0buf):
        W = WCH[i]
        kvi, wvi, tvi, bvi = kv[i], wv[i], tv[i], bv[i]
        syi, mei, spi = sy[i], me[i], sp[i]

        pltpu.make_async_copy(mei, out_slice(mo[i], i), msem[i]).wait()
        pltpu.make_async_copy(spi, out_slice(so[i], i), ssem[i]).wait()

        def jbody(jb, carry):
            o = pl.multiple_of(jb * L, L)
            bb = bvi[pl.ds(o, L)]
            acc = [bb] * NR
            for kp in range(K // 2):
                ab = plsc.bitcast(kvi[kp, pl.ds(o, L)], jnp.int16)
                ia, ib = plsc.unpack(ab, format=ILV)
                for k, idx in ((2 * kp, ia), (2 * kp + 1, ib)):
                    wk = wvi[k, pl.ds(o, L)]
                    if i == 0:
                        g = [plsc.load_gather(x0buf, [idx])]
                        g += [plsc.load_gather(x0buf, [idx + roffv[r - 1]])
                              for r in range(1, NR)]
                    else:
                        g = []
                        for p in range(NP):
                            gp = plsc.load_gather(xpl[i][p], [idx])
                            gb = plsc.bitcast(gp, jnp.bfloat16)
                            g += list(plsc.unpack(gb, format=ILV))
                    for r in range(NR):
                        acc[r] = acc[r] + g[r] * wk
            thrv = tvi[pl.ds(o, L)]
            spks = []
            for r in range(NR):
                m = mei[r, pl.ds(o, L)]
                sn = ALPHA * syi[r, pl.ds(o, L)] + acc[r]
                mn = BETA * m + sn - jnp.where(m > thrv, thrv, 0.0)
                spkv = jnp.where(mn > thrv, 1.0, 0.0)
                syi[r, pl.ds(o, L)] = sn
                mei[r, pl.ds(o, L)] = mn
                spi[r, pl.ds(o, L)] = spkv
                spks.append(spkv)
            if i < 3:
                for p in range(NP):
                    pk = plsc.pack(spks[2 * p], spks[2 * p + 1], format=ILV)
                    spp[i][p][pl.ds(o, L)] = plsc.bitcast(pk, jnp.int32)
            return carry
        lax.fori_loop(0, W // L, jbody, None)

        pltpu.async_copy(mei, mo[i].at[t, pl.ds(b0, NR), pl.ds(q * W, W)],
                         msem[i])
        pltpu.async_copy(spi, so[i].at[t, pl.ds(b0, NR), pl.ds(q * W, W)],
                         ssem[i])

    def stage(s, x0buf, xc, xc_prev):
        pl.when(s < T)(lambda: do_layer(0, s, x0buf))
        wait_reads(xc_prev)
        for i in (1, 2):
            pl.when((s >= i) & (s < T + i))(
                lambda i=i: do_layer(i, s - i, x0buf))
        pl.when((s >= 3) & (s < T + 3) & (q < 4))(
            lambda: do_layer(3, s - 3, x0buf))
        for i in range(3):
            W = WCH[i]
            for p in range(NP):
                pltpu.async_copy(spp[i][p],
                                 xc[i].at[gl * NP + p, pl.ds(q * W, W)],
                                 psem)
        for i in range(3):
            W = WCH[i]
            for p in range(NP):
                pltpu.make_async_copy(spp[i][p],
                                      xc[i].at[gl * NP + p,
                                               pl.ds(q * W, W)],
                                      psem).wait()
        plsc.subcore_barrier()
        fire_reads(xc)

    def pair(pidx, carry):
        s0 = pidx * 2
        fetch_x0(jnp.minimum(s0 + 1, T - 1), x0b, sem_b)
        wait_x0(x0a, sem_a)
        stage(s0, x0a, xcA, xcB)
        fetch_x0(jnp.minimum(s0 + 2, T - 1), x0a, sem_a)
        wait_x0(x0b, sem_b)
        stage(s0 + 1, x0b, xcB, xcA)
        return carry

    lax.fori_loop(0, NSTAGE // 2, pair, None)

    wait_x0(x0a, sem_a)
    wait_reads(xcB)
    for i in range(4):
        def drain(i=i):
            pltpu.make_async_copy(me[i], out_slice(mo[i], i),
                                  msem[i]).wait()
            pltpu.make_async_copy(sp[i], out_slice(so[i], i),
                                  ssem[i]).wait()
        if i == 3:
            pl.when(q < 4)(drain)
        else:
            drain()


@jax.jit
def _run(input, knnT, wT, thr, bias):
    mesh = plsc.VectorSubcoreMesh(core_axis_name="c", subcore_axis_name="s")
    out_type = (
        tuple(jax.ShapeDtypeStruct((T, B, d), jnp.float32) for d in DIMS)
        + tuple(jax.ShapeDtypeStruct((T, B, d), jnp.float32) for d in DIMS)
    )
    scratch = [
        pltpu.VMEM((NR * PREV[0],), jnp.float32),
        pltpu.VMEM((NR * PREV[0],), jnp.float32),
    ]
    for i in (1, 2, 3):
        scratch += [pltpu.VMEM((PREV[i],), jnp.int32)] * NP
    for W in WCH:
        scratch += [
            pltpu.VMEM((K // 2, W), jnp.int32),
            pltpu.VMEM((K, W), jnp.float32),
            pltpu.VMEM((W,), jnp.float32),
            pltpu.VMEM((W,), jnp.float32),
            pltpu.VMEM((NR, W), jnp.float32),
            pltpu.VMEM((NR, W), jnp.float32),
            pltpu.VMEM((NR, W), jnp.float32),
        ]
    for W in WCH[:3]:
        scratch += [pltpu.VMEM((W,), jnp.int32)] * NP
    scratch += [pltpu.VMEM_SHARED((NG_PER_CORE * NP, d), jnp.int32)
                for d in DIMS[:3]] * 2
    scratch += [pltpu.SemaphoreType.DMA] * 12
    flat_in = [input]
    for i in range(4):
        flat_in += [knnT[i], wT[i], thr[i], bias[i]]
    run = pl.kernel(_body, out_type=out_type, mesh=mesh,
                    scratch_types=scratch,
                    compiler_params=pltpu.CompilerParams(
                        needs_layout_passes=False))
    outs = run(*flat_in)
    return outs[:4], outs[4:]


def kernel(input, weight0, bias0, knn0, thr0, weight1, bias1, knn1, thr1,
           weight2, bias2, knn2, thr2, weight3, bias3, knn3, thr3,
           fc_w, fc_b):
    def _pack_idx(k):
        kT = k.T.astype(jnp.int32)
        return kT[0::2] | (kT[1::2] << 16)
    knnT = tuple(_pack_idx(k) for k in (knn0, knn1, knn2, knn3))
    wT = tuple(w.T for w in (weight0, weight1, weight2, weight3))
    thr = (thr0, thr1, thr2, thr3)
    bias = tuple(b.reshape(-1) for b in (bias0, bias1, bias2, bias3))
    mem_rec, spk_rec = _run(input, knnT, wT, thr, bias)
    angles = jnp.dot(mem_rec[3][T - 1], fc_w.T) + fc_b
    return tuple(mem_rec) + tuple(spk_rec) + (angles,)


# speedup vs baseline: 51.8052x; 1.0093x over previous
import jax
import jax.numpy as jnp
from jax import lax
from jax.experimental import pallas as pl
from jax.experimental.pallas import tpu as pltpu
from jax.experimental.pallas import tpu_sc as plsc

B = 16
T = 20
K = 16
DIMS = (4096, 2048, 1024, 512)
PREV = (8192, 4096, 2048, 1024)
ALPHA = 0.9
BETA = 0.85
NQ = 8
NR = 4
NP = NR // 2
NG_PER_CORE = 2
L = 16
ILV = plsc.PackFormat.INTERLEAVED
WCH = (512, 256, 128, 128)
NSTAGE = T + 4


def _body(input_h,
          knnT0, wT0, thr0_h, bias0_h,
          knnT1, wT1, thr1_h, bias1_h,
          knnT2, wT2, thr2_h, bias2_h,
          knnT3, wT3, thr3_h, bias3_h,
          mo0, mo1, mo2, mo3, so0, so1, so2, so3,
          x0a, x0b,
          xp1a, xp1b, xp2a, xp2b, xp3a, xp3b,
          kv0, wv0, tv0, bv0, sy0, me0, sp0,
          kv1, wv1, tv1, bv1, sy1, me1, sp1,
          kv2, wv2, tv2, bv2, sy2, me2, sp2,
          kv3, wv3, tv3, bv3, sy3, me3, sp3,
          spp00, spp01, spp10, spp11, spp20, spp21,
          xcA0, xcA1, xcA2, xcB0, xcB1, xcB2,
          msem0, msem1, msem2, msem3, ssem0, ssem1, ssem2, ssem3,
          sem_a, sem_b, xsem, psem):
    c = lax.axis_index("c")
    s_id = lax.axis_index("s")
    gl = s_id // NQ
    q = s_id % NQ
    b0 = (c * NG_PER_CORE + gl) * NR

    knn_h = (knnT0, knnT1, knnT2, knnT3)
    w_h = (wT0, wT1, wT2, wT3)
    thr_h = (thr0_h, thr1_h, thr2_h, thr3_h)
    bias_h = (bias0_h, bias1_h, bias2_h, bias3_h)
    mo = (mo0, mo1, mo2, mo3)
    so = (so0, so1, so2, so3)
    kv = (kv0, kv1, kv2, kv3)
    wv = (wv0, wv1, wv2, wv3)
    tv = (tv0, tv1, tv2, tv3)
    bv = (bv0, bv1, bv2, bv3)
    sy = (sy0, sy1, sy2, sy3)
    me = (me0, me1, me2, me3)
    sp = (sp0, sp1, sp2, sp3)
    spp = ((spp00, spp01), (spp10, spp11), (spp20, spp21))
    xcA = (xcA0, xcA1, xcA2)
    xcB = (xcB0, xcB1, xcB2)
    xpl = {1: (xp1a, xp1b), 2: (xp2a, xp2b), 3: (xp3a, xp3b)}
    msem = (msem0, msem1, msem2, msem3)
    ssem = (ssem0, ssem1, ssem2, ssem3)

    zeros16 = jnp.zeros((L,), jnp.float32)
    roffv = tuple(jnp.full((L,), r * PREV[0], jnp.int32)
                  for r in range(1, NR))

    def fetch_x0(t, buf, sem):
        for r in range(NR):
            pltpu.async_copy(input_h.at[b0 + r, t],
                             buf.at[pl.ds(r * PREV[0], PREV[0])], sem)

    def wait_x0(buf, sem):
        for r in range(NR):
            pltpu.make_async_copy(input_h.at[b0, 0],
                                  buf.at[pl.ds(r * PREV[0], PREV[0])],
                                  sem).wait()

    def out_slice(o, i):
        W = WCH[i]
        return o.at[0, pl.ds(b0, NR), pl.ds(q * W, W)]

    def fire_reads(xc):
        for i in (1, 2, 3):
            d = PREV[i]
            for p in range(NP):
                pltpu.async_copy(xc[i - 1].at[gl * NP + p, :],
                                 xpl[i][p].at[pl.ds(0, d)], xsem)

    def wait_reads(xc):
        for i in (1, 2, 3):
            d = PREV[i]
            for p in range(NP):
                pltpu.make_async_copy(xc[i - 1].at[gl * NP + p, :],
                                      xpl[i][p].at[pl.ds(0, d)],
                                      xsem).wait()

    for i in range(4):
        W = WCH[i]

        def prolog(i=i, W=W):
            j0 = q * W
            pltpu.sync_copy(knn_h[i].at[:, pl.ds(j0, W)], kv[i])
            pltpu.sync_copy(w_h[i].at[:, pl.ds(j0, W)], wv[i])
            pltpu.sync_copy(thr_h[i].at[pl.ds(j0, W)], tv[i])
            pltpu.sync_copy(bias_h[i].at[pl.ds(j0, W)], bv[i])

            def zbody(jb, carry):
                o = pl.multiple_of(jb * L, L)
                for r in range(NR):
                    sy[i][r, pl.ds(o, L)] = zeros16
                    me[i][r, pl.ds(o, L)] = zeros16
                return carry
            lax.fori_loop(0, W // L, zbody, None)
            pltpu.async_copy(me[i], out_slice(mo[i], i), msem[i])
            pltpu.async_copy(sp[i], out_slice(so[i], i), ssem[i])
        if i == 3:
            pl.when(q < 4)(prolog)
        else:
            prolog()
    fetch_x0(0, x0a, sem_a)
    fire_reads(xcB)

    def do_layer(i, t, x0buf):
        W = WCH[i]
        kvi, wvi, tvi, bvi = kv[i], wv[i], tv[i], bv[i]
        syi, mei, spi = sy[i], me[i], sp[i]

        pltpu.make_async_copy(mei, out_slice(mo[i], i), msem[i]).wait()
        pltpu.make_async_copy(spi, out_slice(so[i], i), ssem[i]).wait()

        def jbody(o):
            o = pl.multiple_of(o, L)
            bb = bvi[pl.ds(o, L)]
            acc = [bb] * NR
            for kp in range(K // 2):
                ab = plsc.bitcast(kvi[kp, pl.ds(o, L)], jnp.int16)
                ia, ib = plsc.unpack(ab, format=ILV)
                for k, idx in ((2 * kp, ia), (2 * kp + 1, ib)):
                    wk = wvi[k, pl.ds(o, L)]
                    if i == 0:
                        g = [plsc.load_gather(x0buf, [idx])]
                        g += [plsc.load_gather(x0buf, [idx + roffv[r - 1]])
                              for r in range(1, NR)]
                    else:
                        g = []
                        for p in range(NP):
                            gp = plsc.load_gather(xpl[i][p], [idx])
                            gb = plsc.bitcast(gp, jnp.bfloat16)
                            g += list(plsc.unpack(gb, format=ILV))
                    for r in range(NR):
                        acc[r] = acc[r] + g[r] * wk
            thrv = tvi[pl.ds(o, L)]
            spks = []
            for r in range(NR):
                m = mei[r, pl.ds(o, L)]
                sn = ALPHA * syi[r, pl.ds(o, L)] + acc[r]
                mn = BETA * m + sn - jnp.where(m > thrv, thrv, 0.0)
                spkv = jnp.where(mn > thrv, 1.0, 0.0)
                syi[r, pl.ds(o, L)] = sn
                mei[r, pl.ds(o, L)] = mn
                spi[r, pl.ds(o, L)] = spkv
                spks.append(spkv)
            if i < 3:
                for p in range(NP):
                    pk = plsc.pack(spks[2 * p], spks[2 * p + 1], format=ILV)
                    spp[i][p][pl.ds(o, L)] = plsc.bitcast(pk, jnp.int32)
        plsc.parallel_loop(0, W, step=L, unroll=2)(jbody)

        pltpu.async_copy(mei, mo[i].at[t, pl.ds(b0, NR), pl.ds(q * W, W)],
                         msem[i])
        pltpu.async_copy(spi, so[i].at[t, pl.ds(b0, NR), pl.ds(q * W, W)],
                         ssem[i])

    def stage(s, x0buf, xc, xc_prev):
        pl.when(s < T)(lambda: do_layer(0, s, x0buf))
        wait_reads(xc_prev)
        for i in (1, 2):
            pl.when((s >= i) & (s < T + i))(
                lambda i=i: do_layer(i, s - i, x0buf))
        pl.when((s >= 3) & (s < T + 3) & (q < 4))(
            lambda: do_layer(3, s - 3, x0buf))
        for i in range(3):
            W = WCH[i]
            for p in range(NP):
                pltpu.async_copy(spp[i][p],
                                 xc[i].at[gl * NP + p, pl.ds(q * W, W)],
                                 psem)
        for i in range(3):
            W = WCH[i]
            for p in range(NP):
                pltpu.make_async_copy(spp[i][p],
                                      xc[i].at[gl * NP + p,
                                               pl.ds(q * W, W)],
                                      psem).wait()
        plsc.subcore_barrier()
        fire_reads(xc)

    def pair(pidx, carry):
        s0 = pidx * 2
        fetch_x0(jnp.minimum(s0 + 1, T - 1), x0b, sem_b)
        wait_x0(x0a, sem_a)
        stage(s0, x0a, xcA, xcB)
        fetch_x0(jnp.minimum(s0 + 2, T - 1), x0a, sem_a)
        wait_x0(x0b, sem_b)
        stage(s0 + 1, x0b, xcB, xcA)
        return carry

    lax.fori_loop(0, NSTAGE // 2, pair, None)

    wait_x0(x0a, sem_a)
    wait_reads(xcB)
    for i in range(4):
        def drain(i=i):
            pltpu.make_async_copy(me[i], out_slice(mo[i], i),
                                  msem[i]).wait()
            pltpu.make_async_copy(sp[i], out_slice(so[i], i),
                                  ssem[i]).wait()
        if i == 3:
            pl.when(q < 4)(drain)
        else:
            drain()


@jax.jit
def _run(input, knnT, wT, thr, bias):
    mesh = plsc.VectorSubcoreMesh(core_axis_name="c", subcore_axis_name="s")
    out_type = (
        tuple(jax.ShapeDtypeStruct((T, B, d), jnp.float32) for d in DIMS)
        + tuple(jax.ShapeDtypeStruct((T, B, d), jnp.float32) for d in DIMS)
    )
    scratch = [
        pltpu.VMEM((NR * PREV[0],), jnp.float32),
        pltpu.VMEM((NR * PREV[0],), jnp.float32),
    ]
    for i in (1, 2, 3):
        scratch += [pltpu.VMEM((PREV[i],), jnp.int32)] * NP
    for W in WCH:
        scratch += [
            pltpu.VMEM((K // 2, W), jnp.int32),
            pltpu.VMEM((K, W), jnp.float32),
            pltpu.VMEM((W,), jnp.float32),
            pltpu.VMEM((W,), jnp.float32),
            pltpu.VMEM((NR, W), jnp.float32),
            pltpu.VMEM((NR, W), jnp.float32),
            pltpu.VMEM((NR, W), jnp.float32),
        ]
    for W in WCH[:3]:
        scratch += [pltpu.VMEM((W,), jnp.int32)] * NP
    scratch += [pltpu.VMEM_SHARED((NG_PER_CORE * NP, d), jnp.int32)
                for d in DIMS[:3]] * 2
    scratch += [pltpu.SemaphoreType.DMA] * 12
    flat_in = [input]
    for i in range(4):
        flat_in += [knnT[i], wT[i], thr[i], bias[i]]
    run = pl.kernel(_body, out_type=out_type, mesh=mesh,
                    scratch_types=scratch,
                    compiler_params=pltpu.CompilerParams(
                        needs_layout_passes=False))
    outs = run(*flat_in)
    return outs[:4], outs[4:]


def kernel(input, weight0, bias0, knn0, thr0, weight1, bias1, knn1, thr1,
           weight2, bias2, knn2, thr2, weight3, bias3, knn3, thr3,
           fc_w, fc_b):
    def _pack_idx(k):
        kT = k.T.astype(jnp.int32)
        return kT[0::2] | (kT[1::2] << 16)
    knnT = tuple(_pack_idx(k) for k in (knn0, knn1, knn2, knn3))
    wT = tuple(w.T for w in (weight0, weight1, weight2, weight3))
    thr = (thr0, thr1, thr2, thr3)
    bias = tuple(b.reshape(-1) for b in (bias0, bias1, bias2, bias3))
    mem_rec, spk_rec = _run(input, knnT, wT, thr, bias)
    angles = jnp.dot(mem_rec[3][T - 1], fc_w.T) + fc_b
    return tuple(mem_rec) + tuple(spk_rec) + (angles,)


# speedup vs baseline: 53.2928x; 1.0287x over previous
import jax
import jax.numpy as jnp
from jax import lax
from jax.experimental import pallas as pl
from jax.experimental.pallas import tpu as pltpu
from jax.experimental.pallas import tpu_sc as plsc

B = 16
T = 20
K = 16
DIMS = (4096, 2048, 1024, 512)
PREV = (8192, 4096, 2048, 1024)
ALPHA = 0.9
BETA = 0.85
NQ = 8
NR = 4
NP = NR // 2
NG_PER_CORE = 2
L = 16
ILV = plsc.PackFormat.INTERLEAVED
WCH = (512, 256, 128, 128)
NSTAGE = T + 4


def _body(input_h,
          knnP_all, wT_all,
          thr0_h, bias0_h, thr1_h, bias1_h,
          thr2_h, bias2_h, thr3_h, bias3_h,
          mo0, mo1, mo2, mo3, so0, so1, so2, so3,
          x0a, x0b,
          xp1a, xp1b, xp2a, xp2b, xp3a, xp3b,
          kv0, wv0, tv0, bv0, sy0, me0, sp0,
          kv1, wv1, tv1, bv1, sy1, me1, sp1,
          kv2, wv2, tv2, bv2, sy2, me2, sp2,
          kv3, wv3, tv3, bv3, sy3, me3, sp3,
          spp00, spp01, spp10, spp11, spp20, spp21,
          xcA0, xcA1, xcA2, xcB0, xcB1, xcB2,
          msem0, msem1, msem2, msem3, ssem0, ssem1, ssem2, ssem3,
          sem_a, sem_b, xsem, psem):
    c = lax.axis_index("c")
    s_id = lax.axis_index("s")
    gl = s_id // NQ
    q = s_id % NQ
    b0 = (c * NG_PER_CORE + gl) * NR

    thr_h = (thr0_h, thr1_h, thr2_h, thr3_h)
    bias_h = (bias0_h, bias1_h, bias2_h, bias3_h)
    mo = (mo0, mo1, mo2, mo3)
    so = (so0, so1, so2, so3)
    kv = (kv0, kv1, kv2, kv3)
    wv = (wv0, wv1, wv2, wv3)
    tv = (tv0, tv1, tv2, tv3)
    bv = (bv0, bv1, bv2, bv3)
    sy = (sy0, sy1, sy2, sy3)
    me = (me0, me1, me2, me3)
    sp = (sp0, sp1, sp2, sp3)
    spp = ((spp00, spp01), (spp10, spp11), (spp20, spp21))
    xcA = (xcA0, xcA1, xcA2)
    xcB = (xcB0, xcB1, xcB2)
    xpl = {1: (xp1a, xp1b), 2: (xp2a, xp2b), 3: (xp3a, xp3b)}
    msem = (msem0, msem1, msem2, msem3)
    ssem = (ssem0, ssem1, ssem2, ssem3)

    zeros16 = jnp.zeros((L,), jnp.float32)
    roffv = tuple(jnp.full((L,), r * PREV[0], jnp.int32)
                  for r in range(1, NR))

    def fetch_x0(t, buf, sem):
        for r in range(NR):
            pltpu.async_copy(input_h.at[b0 + r, t],
                             buf.at[pl.ds(r * PREV[0], PREV[0])], sem)

    def wait_x0(buf, sem):
        for r in range(NR):
            pltpu.make_async_copy(input_h.at[b0, 0],
                                  buf.at[pl.ds(r * PREV[0], PREV[0])],
                                  sem).wait()

    def out_slice(o, i):
        W = WCH[i]
        return o.at[0, pl.ds(b0, NR), pl.ds(q * W, W)]

    def fire_reads(xc):
        for i in (1, 2, 3):
            d = PREV[i]
            for p in range(NP):
                pltpu.async_copy(xc[i - 1].at[gl * NP + p, :],
                                 xpl[i][p].at[pl.ds(0, d)], xsem)

    def wait_reads(xc):
        for i in (1, 2, 3):
            d = PREV[i]
            for p in range(NP):
                pltpu.make_async_copy(xc[i - 1].at[gl * NP + p, :],
                                      xpl[i][p].at[pl.ds(0, d)],
                                      xsem).wait()

    for i in range(4):
        W = WCH[i]

        loff = sum(DIMS[:i])

        def prolog(i=i, W=W, loff=loff):
            j0 = q * W
            jt = loff + j0
            pltpu.sync_copy(knnP_all.at[:, pl.ds(jt, W)], kv[i])
            pltpu.sync_copy(wT_all.at[:, pl.ds(jt, W)], wv[i])
            pltpu.sync_copy(thr_h[i].at[pl.ds(j0, W)], tv[i])
            pltpu.sync_copy(bias_h[i].at[pl.ds(j0, W)], bv[i])

            def zbody(jb, carry):
                o = pl.multiple_of(jb * L, L)
                for r in range(NR):
                    sy[i][r, pl.ds(o, L)] = zeros16
                    me[i][r, pl.ds(o, L)] = zeros16
                return carry
            lax.fori_loop(0, W // L, zbody, None)
            pltpu.async_copy(me[i], out_slice(mo[i], i), msem[i])
            pltpu.async_copy(sp[i], out_slice(so[i], i), ssem[i])
        if i == 3:
            pl.when(q < 4)(prolog)
        else:
            prolog()
    fetch_x0(0, x0a, sem_a)
    fire_reads(xcB)

    def do_layer(i, t, x0buf):
        W = WCH[i]
        kvi, wvi, tvi, bvi = kv[i], wv[i], tv[i], bv[i]
        syi, mei, spi = sy[i], me[i], sp[i]

        pltpu.make_async_copy(mei, out_slice(mo[i], i), msem[i]).wait()
        pltpu.make_async_copy(spi, out_slice(so[i], i), ssem[i]).wait()

        def jbody(o):
            o = pl.multiple_of(o, L)
            bb = bvi[pl.ds(o, L)]
            acc = [bb] * NR
            for kp in range(K // 2):
                ab = plsc.bitcast(kvi[kp, pl.ds(o, L)], jnp.int16)
                ia, ib = plsc.unpack(ab, format=ILV)
                for k, idx in ((2 * kp, ia), (2 * kp + 1, ib)):
                    wk = wvi[k, pl.ds(o, L)]
                    if i == 0:
                        g = [plsc.load_gather(x0buf, [idx])]
                        g += [plsc.load_gather(x0buf, [idx + roffv[r - 1]])
                              for r in range(1, NR)]
                    else:
                        g = []
                        for p in range(NP):
                            gp = plsc.load_gather(xpl[i][p], [idx])
                            gb = plsc.bitcast(gp, jnp.bfloat16)
                            g += list(plsc.unpack(gb, format=ILV))
                    for r in range(NR):
                        acc[r] = acc[r] + g[r] * wk
            thrv = tvi[pl.ds(o, L)]
            spks = []
            for r in range(NR):
                m = mei[r, pl.ds(o, L)]
                sn = ALPHA * syi[r, pl.ds(o, L)] + acc[r]
                mn = BETA * m + sn - jnp.where(m > thrv, thrv, 0.0)
                spkv = jnp.where(mn > thrv, 1.0, 0.0)
                syi[r, pl.ds(o, L)] = sn
                mei[r, pl.ds(o, L)] = mn
                spi[r, pl.ds(o, L)] = spkv
                spks.append(spkv)
            if i < 3:
                for p in range(NP):
                    pk = plsc.pack(spks[2 * p], spks[2 * p + 1], format=ILV)
                    spp[i][p][pl.ds(o, L)] = plsc.bitcast(pk, jnp.int32)
        plsc.parallel_loop(0, W, step=L, unroll=2)(jbody)

        pltpu.async_copy(mei, mo[i].at[t, pl.ds(b0, NR), pl.ds(q * W, W)],
                         msem[i])
        pltpu.async_copy(spi, so[i].at[t, pl.ds(b0, NR), pl.ds(q * W, W)],
                         ssem[i])

    def stage(s, x0buf, xc, xc_prev):
        pl.when(s < T)(lambda: do_layer(0, s, x0buf))
        wait_reads(xc_prev)
        for i in (1, 2):
            pl.when((s >= i) & (s < T + i))(
                lambda i=i: do_layer(i, s - i, x0buf))
        pl.when((s >= 3) & (s < T + 3) & (q < 4))(
            lambda: do_layer(3, s - 3, x0buf))
        for i in range(3):
            W = WCH[i]
            for p in range(NP):
                pltpu.async_copy(spp[i][p],
                                 xc[i].at[gl * NP + p, pl.ds(q * W, W)],
                                 psem)
        for i in range(3):
            W = WCH[i]
            for p in range(NP):
                pltpu.make_async_copy(spp[i][p],
                                      xc[i].at[gl * NP + p,
                                               pl.ds(q * W, W)],
                                      psem).wait()
        plsc.subcore_barrier()
        fire_reads(xc)

    def pair(pidx, carry):
        s0 = pidx * 2
        fetch_x0(jnp.minimum(s0 + 1, T - 1), x0b, sem_b)
        wait_x0(x0a, sem_a)
        stage(s0, x0a, xcA, xcB)
        fetch_x0(jnp.minimum(s0 + 2, T - 1), x0a, sem_a)
        wait_x0(x0b, sem_b)
        stage(s0 + 1, x0b, xcB, xcA)
        return carry

    lax.fori_loop(0, NSTAGE // 2, pair, None)

    wait_x0(x0a, sem_a)
    wait_reads(xcB)
    for i in range(4):
        def drain(i=i):
            pltpu.make_async_copy(me[i], out_slice(mo[i], i),
                                  msem[i]).wait()
            pltpu.make_async_copy(sp[i], out_slice(so[i], i),
                                  ssem[i]).wait()
        if i == 3:
            pl.when(q < 4)(drain)
        else:
            drain()


@jax.jit
def _run(input, knnT, wT, thr, bias):
    mesh = plsc.VectorSubcoreMesh(core_axis_name="c", subcore_axis_name="s")
    out_type = (
        tuple(jax.ShapeDtypeStruct((T, B, d), jnp.float32) for d in DIMS)
        + tuple(jax.ShapeDtypeStruct((T, B, d), jnp.float32) for d in DIMS)
    )
    scratch = [
        pltpu.VMEM((NR * PREV[0],), jnp.float32),
        pltpu.VMEM((NR * PREV[0],), jnp.float32),
    ]
    for i in (1, 2, 3):
        scratch += [pltpu.VMEM((PREV[i],), jnp.int32)] * NP
    for W in WCH:
        scratch += [
            pltpu.VMEM((K // 2, W), jnp.int32),
            pltpu.VMEM((K, W), jnp.float32),
            pltpu.VMEM((W,), jnp.float32),
            pltpu.VMEM((W,), jnp.float32),
            pltpu.VMEM((NR, W), jnp.float32),
            pltpu.VMEM((NR, W), jnp.float32),
            pltpu.VMEM((NR, W), jnp.float32),
        ]
    for W in WCH[:3]:
        scratch += [pltpu.VMEM((W,), jnp.int32)] * NP
    scratch += [pltpu.VMEM_SHARED((NG_PER_CORE * NP, d), jnp.int32)
                for d in DIMS[:3]] * 2
    scratch += [pltpu.SemaphoreType.DMA] * 12
    flat_in = [input, knnT, wT]
    for i in range(4):
        flat_in += [thr[i], bias[i]]
    run = pl.kernel(_body, out_type=out_type, mesh=mesh,
                    scratch_types=scratch,
                    compiler_params=pltpu.CompilerParams(
                        needs_layout_passes=False))
    outs = run(*flat_in)
    return outs[:4], outs[4:]


def kernel(input, weight0, bias0, knn0, thr0, weight1, bias1, knn1, thr1,
           weight2, bias2, knn2, thr2, weight3, bias3, knn3, thr3,
           fc_w, fc_b):
    kT = jnp.concatenate([knn0, knn1, knn2, knn3], axis=0).T.astype(jnp.int32)
    knnT = kT[0::2] | (kT[1::2] << 16)
    wT = jnp.concatenate([weight0, weight1, weight2, weight3], axis=0).T
    thr = (thr0, thr1, thr2, thr3)
    bias = tuple(b.reshape(-1) for b in (bias0, bias1, bias2, bias3))
    mem_rec, spk_rec = _run(input, knnT, wT, thr, bias)
    angles = jnp.dot(mem_rec[3][T - 1], fc_w.T) + fc_b
    return tuple(mem_rec) + tuple(spk_rec) + (angles,)
